# baseline (speedup 1.0000x reference)
import jax
import jax.numpy as jnp
import numpy as np
from jax import lax
from jax.experimental import pallas as pl
from jax.experimental.pallas import tpu as pltpu
from jax.experimental.pallas import tpu_sc as plsc

N = 10000
E = 320000
D = 128
EPS = 1e-5

NC = 2
NS = 16
L = 128
IB = 16
NB = 10
CHUNKS = IB * NB
EPT = CHUNKS * L
EP = NS * EPT
PAD = EP - E
NP = 10112
RPT = NP // NS
CW = 16
DA = D + CW
PKW = D // 2

BLK = 1000

_PERM = np.empty((D,), np.int32)
for _j in range(4):
    for _k in range(16):
        _PERM[32 * _j + 2 * _k] = 16 * _j + _k
        _PERM[32 * _j + 2 * _k + 1] = 64 + 16 * _j + _k


def _sc_body(xn, xc, snn, dnn, scn, dcn, o_nn, o_cn,
             acc, sidx, didx, pk0, pk1, fb, g0, g1, sc):
    c = lax.axis_index("c")
    s = lax.axis_index("s")
    base = s * RPT
    zero16 = jnp.zeros((16,), jnp.float32)
    one16 = jnp.ones((16,), jnp.float32)

    @pl.loop(0, L)
    def _(i):
        for j in range(DA // 16):
            fb[i, pl.ds(16 * j, 16)] = zero16

    for k in range(RPT // L):
        pltpu.sync_copy(fb, acc.at[pl.ds(base + k * L, L)])
    tail = RPT % L
    if tail:
        pltpu.sync_copy(fb.at[pl.ds(0, tail)],
                        acc.at[pl.ds(base + (RPT // L) * L, tail)])

    @pl.loop(0, L)
    def _(i):
        fb[i, pl.ds(D, CW)] = one16

    plsc.subcore_barrier()

    def convert(pk):
        @pl.loop(0, L)
        def _(i):
            for j in range(4):
                v = pk[i, pl.ds(16 * j, 16)]
                fb[i, pl.ds(16 * j, 16)] = lax.bitcast_convert_type(
                    v << 16, jnp.float32)
                fb[i, pl.ds(64 + 16 * j, 16)] = lax.bitcast_convert_type(
                    v & jnp.int32(-65536), jnp.float32)

    def run_type(src_r, dst_r, x_r):
        def wait_scatter():
            pltpu.make_async_copy(fb, acc.at[didx.at[0]], sc).wait()

        @pl.loop(0, NB)
        def _(b):
            pltpu.sync_copy(src_r.at[s, pl.ds(b * IB, IB)], sidx)
            pltpu.sync_copy(dst_r.at[s, pl.ds(b * IB, IB)], didx)
            pltpu.async_copy(x_r.at[sidx.at[0]], pk0, g0)

            @pl.loop(0, IB // 2)
            def _(h):
                a = 2 * h
                pltpu.make_async_copy(x_r.at[sidx.at[a]], pk0, g0).wait()
                pltpu.async_copy(x_r.at[sidx.at[a + 1]], pk1, g1)

                @pl.when(h > 0)
                def _():
                    wait_scatter()

                convert(pk0)
                pltpu.async_copy(fb, acc.at[didx.at[a]], sc, add=True)
                pltpu.make_async_copy(x_r.at[sidx.at[a + 1]], pk1, g1).wait()

                @pl.when(h < IB // 2 - 1)
                def _():
                    pltpu.async_copy(x_r.at[sidx.at[a + 2]], pk0, g0)

                wait_scatter()
                convert(pk1)
                pltpu.async_copy(fb, acc.at[didx.at[a + 1]], sc, add=True)

            wait_scatter()

    @pl.when(c == 0)
    def _():
        run_type(snn, dnn, xn)

    @pl.when(c == 1)
    def _():
        run_type(scn, dcn, xc)

    plsc.subcore_barrier()

    @pl.when(c == 0)
    def _():
        pltpu.sync_copy(acc.at[pl.ds(base, RPT)], o_nn.at[pl.ds(base, RPT)])

    @pl.when(c == 1)
    def _():
        pltpu.sync_copy(acc.at[pl.ds(base, RPT)], o_cn.at[pl.ds(base, RPT)])


_sc_aggregate = pl.kernel(
    _sc_body,
    out_type=(
        jax.ShapeDtypeStruct((NP, DA), jnp.float32),
        jax.ShapeDtypeStruct((NP, DA), jnp.float32),
    ),
    mesh=plsc.VectorSubcoreMesh(core_axis_name="c", subcore_axis_name="s",
                                num_cores=NC, num_subcores=NS),
    scratch_types=[
        pltpu.VMEM_SHARED((NP, DA), jnp.float32),
        pltpu.VMEM((IB, L), jnp.int32),
        pltpu.VMEM((IB, L), jnp.int32),
        pltpu.VMEM((L, PKW), jnp.int32),
        pltpu.VMEM((L, PKW), jnp.int32),
        pltpu.VMEM((L, DA), jnp.float32),
        pltpu.SemaphoreType.DMA,
        pltpu.SemaphoreType.DMA,
        pltpu.SemaphoreType.DMA,
    ],
    compiler_params=pltpu.CompilerParams(use_tc_tiling_on_sc=False),
)


def _tc_body(x, snn, scn, wlnn, wlcn, wrnn, wrcn, bnn, bcn, lnw, lnb, out):
    aggn = snn[:, :D] / jnp.maximum(snn[:, D:D + 1], 1.0)
    aggc = scn[:, :D] / jnp.maximum(scn[:, D:D + 1], 1.0)
    h = (jnp.dot(aggn, wlnn[:], preferred_element_type=jnp.float32)
         + jnp.dot(aggc, wlcn[:], preferred_element_type=jnp.float32)
         + jnp.dot(x[:], wrnn[:] + wrcn[:], preferred_element_type=jnp.float32)
         + bnn[:] + bcn[:])
    h = jnp.maximum(h, 0.0)
    mu = jnp.mean(h, axis=1, keepdims=True)
    d = h - mu
    var = jnp.mean(d * d, axis=1, keepdims=True)
    out[:] = d * lax.rsqrt(var + EPS) * lnw[:] + lnb[:]


_row_spec = pl.BlockSpec((BLK, D), lambda i: (i, 0))
_agg_spec = pl.BlockSpec((BLK, DA), lambda i: (i, 0))
_w_spec = pl.BlockSpec((D, D), lambda i: (0, 0))
_v_spec = pl.BlockSpec((1, D), lambda i: (0, 0))

_tc_fuse = pl.pallas_call(
    _tc_body,
    grid=(N // BLK,),
    in_specs=[_row_spec, _agg_spec, _agg_spec,
              _w_spec, _w_spec, _w_spec, _w_spec,
              _v_spec, _v_spec, _v_spec, _v_spec],
    out_specs=_row_spec,
    out_shape=jax.ShapeDtypeStruct((N, D), jnp.float32),
)


def _prep_edges(ei):
    src = ei[0].astype(jnp.int32)
    dst = ei[1].astype(jnp.int32)
    src = jnp.concatenate([src, jnp.zeros((PAD,), jnp.int32)])
    dst = jnp.concatenate([dst, jnp.full((PAD,), N, jnp.int32)])
    return src.reshape(NS, CHUNKS, L), dst.reshape(NS, CHUNKS, L)


def _pack_x(x):
    xb = x[:, _PERM].astype(jnp.bfloat16)
    return jax.lax.bitcast_convert_type(xb.reshape(N, PKW, 2), jnp.int32)


def kernel(x_node, x_ctx, edge_index_nn, edge_index_cn,
           Wl_nn, Wr_nn, b_nn, Wl_cn, Wr_cn, b_cn, ln_w, ln_b):
    snn, dnn = _prep_edges(edge_index_nn)
    scn, dcn = _prep_edges(edge_index_cn)
    s_nn, s_cn = _sc_aggregate(_pack_x(x_node), _pack_x(x_ctx),
                               snn, dnn, scn, dcn)
    return _tc_fuse(x_node, s_nn, s_cn,
                    Wl_nn, Wl_cn, Wr_nn, Wr_cn,
                    b_nn.reshape(1, D), b_cn.reshape(1, D),
                    ln_w.reshape(1, D), ln_b.reshape(1, D))
